# baseline (speedup 1.0000x reference)
import jax
import jax.numpy as jnp
from jax import lax
from jax.experimental import pallas as pl
from jax.experimental.pallas import tpu as pltpu
from jax.experimental.pallas import tpu_sc as plsc

_N = 10000
_NP = 10240
_E = 320000
_EP = _E + 512
_D = 128
_NT = 16
_CH = 32
_EPT = _E // _NT
_NCH = _EPT // _CH
_NCH2 = _EP // _CH // (2 * _NT)
_RPT = _NP // _NT

_f32 = jnp.float32
_i32 = jnp.int32

_mesh = plsc.VectorSubcoreMesh(core_axis_name="c", subcore_axis_name="s")


def _front_body(h_ref, wfc_ref, wa1_ref, zzw_ref, ewr_ref):
    z = lax.dot_general(h_ref[...], wfc_ref[...],
                        (((1,), (1,)), ((), ())),
                        preferred_element_type=_f32)
    p = lax.dot_general(z, wa1_ref[...], (((1,), (1,)), ((), ())),
                        preferred_element_type=_f32)
    ew = jnp.exp(p - jnp.max(p))
    zzw_ref[:_N, :] = z * ew
    zzw_ref[_N:, :] = z
    ewr_ref[...] = jnp.broadcast_to(ew, (_N, _D))


def _front(h, W_fc, wa1):
    return pl.pallas_call(
        _front_body,
        out_shape=[jax.ShapeDtypeStruct((2 * _N, _D), _f32),
                   jax.ShapeDtypeStruct((_N, _D), _f32)],
    )(h, W_fc, wa1)


def _ug_body(zzw_hbm, src_hbm, dst_hbm, ug_hbm, src_a, dst_a, rows_a,
             src_b, dst_b, rows_b, acc_s, sem_a, sem_b, ssem_a, ssem_b):
    c = lax.axis_index("c")
    s = lax.axis_index("s")
    z16 = jnp.zeros((16,), _f32)
    row0 = s * _RPT
    for i in range(_CH):
        for j in range(_D // 16):
            rows_a[i, pl.ds(j * 16, 16)] = z16
    for kk in range(_RPT // _CH):
        pltpu.sync_copy(rows_a, acc_s.at[pl.ds(row0 + kk * _CH, _CH)])
    plsc.subcore_barrier()
    off = c * _N
    base0 = s * _EPT

    def _fetch(chunk, src_v, dst_v, rows_v, gsem, ssem, first=False):
        if not first:
            pltpu.make_async_copy(rows_v, acc_s.at[dst_v], ssem).wait()
        base = base0 + chunk * _CH
        pltpu.sync_copy(src_hbm.at[pl.ds(base, _CH)], src_v)
        pltpu.sync_copy(dst_hbm.at[pl.ds(base, _CH)], dst_v)
        for g in range(_CH // 16):
            src_v[pl.ds(g * 16, 16)] = src_v[pl.ds(g * 16, 16)] + off
        pltpu.async_copy(zzw_hbm.at[src_v], rows_v, gsem)

    def _drain(src_v, dst_v, rows_v, gsem, ssem):
        pltpu.make_async_copy(zzw_hbm.at[src_v], rows_v, gsem).wait()
        pltpu.async_copy(rows_v, acc_s.at[dst_v], ssem, add=True)

    def _wait_scatter(dst_v, rows_v, ssem):
        pltpu.make_async_copy(rows_v, acc_s.at[dst_v], ssem).wait()

    _fetch(0, src_a, dst_a, rows_a, sem_a, ssem_a, first=True)
    _fetch(1, src_b, dst_b, rows_b, sem_b, ssem_b, first=True)

    def _chunk2(k2, carry):
        _drain(src_a, dst_a, rows_a, sem_a, ssem_a)
        _drain(src_b, dst_b, rows_b, sem_b, ssem_b)
        _fetch(2 * k2 + 2, src_a, dst_a, rows_a, sem_a, ssem_a)
        _fetch(2 * k2 + 3, src_b, dst_b, rows_b, sem_b, ssem_b)
        return carry

    lax.fori_loop(0, (_NCH - 3) // 2, _chunk2, None)
    _drain(src_a, dst_a, rows_a, sem_a, ssem_a)
    _drain(src_b, dst_b, rows_b, sem_b, ssem_b)
    _fetch(_NCH - 1, src_a, dst_a, rows_a, sem_a, ssem_a)
    _drain(src_a, dst_a, rows_a, sem_a, ssem_a)
    _wait_scatter(dst_a, rows_a, ssem_a)
    _wait_scatter(dst_b, rows_b, ssem_b)
    plsc.subcore_barrier()
    pltpu.sync_copy(acc_s.at[pl.ds(row0, _RPT)],
                    ug_hbm.at[pl.ds(c * _NP + row0, _RPT)])


def _ug_edge(zzw, src, dst):
    kern = pl.kernel(
        _ug_body,
        out_type=[jax.ShapeDtypeStruct((2 * _NP, _D), _f32)],
        mesh=_mesh,
        scratch_types=[
            pltpu.VMEM((_CH,), _i32),
            pltpu.VMEM((_CH,), _i32),
            pltpu.VMEM((_CH, _D), _f32),
            pltpu.VMEM((_CH,), _i32),
            pltpu.VMEM((_CH,), _i32),
            pltpu.VMEM((_CH, _D), _f32),
            pltpu.VMEM_SHARED((_NP, _D), _f32),
            pltpu.SemaphoreType.DMA,
            pltpu.SemaphoreType.DMA,
            pltpu.SemaphoreType.DMA,
            pltpu.SemaphoreType.DMA,
        ],
    )
    return kern(zzw, src, dst)


def _sd_body(ewr_hbm, src_hbm, dst_hbm, sd2_hbm, src_a, dst_a, wrow_a,
             src_b, dst_b, wrow_b, sd_a, sd_b, sds_s, sem_a, sem_b,
             ssem_a, ssem_b):
    c = lax.axis_index("c")
    s = lax.axis_index("s")
    z16 = jnp.zeros((16,), _f32)
    iota16 = lax.iota(_i32, 16)
    o01 = jnp.where(iota16 == 1, 1.0, 0.0).astype(_f32)
    row0 = s * _RPT
    for i in range(_CH):
        for j in range(_D // 16):
            sd_a[i, pl.ds(j * 16, 16)] = z16
            sd_b[i, pl.ds(j * 16, 16)] = z16
    for kk in range(_RPT // _CH):
        pltpu.sync_copy(sd_a, sds_s.at[pl.ds(row0 + kk * _CH, _CH)])
    plsc.subcore_barrier()

    def _fetch(chunk, src_v, dst_v, wrow_v, sd_v, gsem, ssem, first=False):
        if not first:
            pltpu.make_async_copy(sd_v, sds_s.at[dst_v], ssem).wait()
        base = ((s * _NCH2 + chunk) * 2 + c) * _CH
        pltpu.sync_copy(src_hbm.at[pl.ds(base, _CH)], src_v)
        pltpu.sync_copy(dst_hbm.at[pl.ds(base, _CH)], dst_v)
        pltpu.async_copy(ewr_hbm.at[src_v], wrow_v, gsem)

    def _drain(src_v, dst_v, wrow_v, sd_v, gsem, ssem):
        pltpu.make_async_copy(ewr_hbm.at[src_v], wrow_v, gsem).wait()
        for r in range(_CH):
            wb = wrow_v[r, pl.ds(0, 16)]
            sd_v[r, pl.ds(0, 16)] = jnp.where(iota16 == 0, wb, o01)
        pltpu.async_copy(sd_v, sds_s.at[dst_v], ssem, add=True)

    def _wait_scatter(dst_v, sd_v, ssem):
        pltpu.make_async_copy(sd_v, sds_s.at[dst_v], ssem).wait()

    _fetch(0, src_a, dst_a, wrow_a, sd_a, sem_a, ssem_a, first=True)
    _fetch(1, src_b, dst_b, wrow_b, sd_b, sem_b, ssem_b, first=True)

    def _chunk2(k2, carry):
        _drain(src_a, dst_a, wrow_a, sd_a, sem_a, ssem_a)
        _drain(src_b, dst_b, wrow_b, sd_b, sem_b, ssem_b)
        _fetch(2 * k2 + 2, src_a, dst_a, wrow_a, sd_a, sem_a, ssem_a)
        _fetch(2 * k2 + 3, src_b, dst_b, wrow_b, sd_b, sem_b, ssem_b)
        return carry

    lax.fori_loop(0, (_NCH2 - 3) // 2, _chunk2, None)
    _drain(src_a, dst_a, wrow_a, sd_a, sem_a, ssem_a)
    _drain(src_b, dst_b, wrow_b, sd_b, sem_b, ssem_b)
    _fetch(_NCH2 - 1, src_a, dst_a, wrow_a, sd_a, sem_a, ssem_a)
    _drain(src_a, dst_a, wrow_a, sd_a, sem_a, ssem_a)
    _wait_scatter(dst_a, sd_a, ssem_a)
    _wait_scatter(dst_b, sd_b, ssem_b)
    plsc.subcore_barrier()
    pltpu.sync_copy(sds_s.at[pl.ds(row0, _RPT)],
                    sd2_hbm.at[pl.ds(c * _NP + row0, _RPT)])


def _sd_edge(ewr, src_p, dst_p):
    kern = pl.kernel(
        _sd_body,
        out_type=[jax.ShapeDtypeStruct((2 * _NP, _D), _f32)],
        mesh=_mesh,
        scratch_types=[
            pltpu.VMEM((_CH,), _i32),
            pltpu.VMEM((_CH,), _i32),
            pltpu.VMEM((_CH, _D), _f32),
            pltpu.VMEM((_CH,), _i32),
            pltpu.VMEM((_CH,), _i32),
            pltpu.VMEM((_CH, _D), _f32),
            pltpu.VMEM((_CH, _D), _f32),
            pltpu.VMEM((_CH, _D), _f32),
            pltpu.VMEM_SHARED((_NP, _D), _f32),
            pltpu.SemaphoreType.DMA,
            pltpu.SemaphoreType.DMA,
            pltpu.SemaphoreType.DMA,
            pltpu.SemaphoreType.DMA,
        ],
    )
    return kern(ewr, src_p, dst_p)


def _comb_body(ug_ref, sd2_ref, zzw_ref, we1_ref, we2_ref, out_ref):
    sden = sd2_ref[:_N, 0:1] + sd2_ref[_NP:_NP + _N, 0:1]
    deg = sd2_ref[:_N, 1:2] + sd2_ref[_NP:_NP + _N, 1:2]
    u = ug_ref[:_N, :]
    gacc = ug_ref[_NP:_NP + _N, :]
    z = zzw_ref[_N:, :]
    pos = sden > 0.0
    h1 = jnp.where(pos, u / jnp.where(pos, sden, 1.0), 0.0)
    h2 = lax.dot_general(gacc, we1_ref[...],
                         (((1,), (1,)), ((), ())),
                         preferred_element_type=_f32)
    h2 = h2 + lax.dot_general(deg * z, we2_ref[...],
                              (((1,), (1,)), ((), ())),
                              preferred_element_type=_f32)
    out_ref[...] = h1 + h2


def _combine(ug, sd2, zzw, we1, we2):
    return pl.pallas_call(
        _comb_body,
        out_shape=jax.ShapeDtypeStruct((_N, _D), _f32),
    )(ug, sd2, zzw, we1, we2)


@jax.jit
def kernel(h, W_fc, W_attn, W_edge, edge_index):
    src = edge_index[0].astype(_i32)
    dst = edge_index[1].astype(_i32)
    src_p = jnp.concatenate([src, jnp.zeros((_EP - _E,), _i32)])
    dst_p = jnp.concatenate([dst, jnp.full((_EP - _E,), _N, _i32)])
    zzw, ewr = _front(h, W_fc, W_attn[:, :_D])
    ug = _ug_edge(zzw, src_p, dst_p)[0]
    sd2 = _sd_edge(ewr, src_p, dst_p)[0]
    return _combine(ug, sd2, zzw, W_edge[:, :_D], W_edge[:, _D:])


# speedup vs baseline: 1.4692x; 1.4692x over previous
import jax
import jax.numpy as jnp
from jax import lax
from jax.experimental import pallas as pl
from jax.experimental.pallas import tpu as pltpu
from jax.experimental.pallas import tpu_sc as plsc

_N = 10000
_NP = 10112
_E = 320000
_EPM = 320256
_EP2 = 321024
_D = 128
_NT = 16
_CH = 48
_EPT = _EPM // _NT
_NCH = _EPT // _CH
_NCH2 = _EP2 // _CH // (2 * _NT)
_RPT = _NP // _NT

_f32 = jnp.float32
_i32 = jnp.int32

_mesh = plsc.VectorSubcoreMesh(core_axis_name="c", subcore_axis_name="s")


def _front_body(h_ref, wfc_ref, wa1_ref, zzw_ref, ewr_ref):
    z = lax.dot_general(h_ref[...], wfc_ref[...],
                        (((1,), (1,)), ((), ())),
                        preferred_element_type=_f32)
    p = lax.dot_general(z, wa1_ref[...], (((1,), (1,)), ((), ())),
                        preferred_element_type=_f32)
    ew = jnp.exp(p - jnp.max(p))
    zzw_ref[:_N, :] = z * ew
    zzw_ref[_N:, :] = z
    ewr_ref[...] = jnp.broadcast_to(ew, (_N, _D))


def _front(h, W_fc, wa1):
    return pl.pallas_call(
        _front_body,
        out_shape=[jax.ShapeDtypeStruct((2 * _N, _D), _f32),
                   jax.ShapeDtypeStruct((_N, _D), _f32)],
    )(h, W_fc, wa1)


def _ug_body(zzw_hbm, zeros_hbm, src_hbm, dst_hbm, ug_hbm, src_a, dst_a,
             rows_a, src_b, dst_b, rows_b, acc_s, sem_a, sem_b):
    c = lax.axis_index("c")
    s = lax.axis_index("s")
    row0 = s * _RPT
    pltpu.sync_copy(zeros_hbm.at[pl.ds(row0, _RPT)],
                    acc_s.at[pl.ds(row0, _RPT)])
    plsc.subcore_barrier()
    off = c * _N
    base0 = s * _EPT

    def _fetch(chunk, src_v, dst_v, rows_v, sem):
        base = base0 + chunk * _CH
        pltpu.sync_copy(src_hbm.at[pl.ds(base, _CH)], src_v)
        pltpu.sync_copy(dst_hbm.at[pl.ds(base, _CH)], dst_v)
        for g in range(_CH // 16):
            src_v[pl.ds(g * 16, 16)] = src_v[pl.ds(g * 16, 16)] + off
        pltpu.async_copy(zzw_hbm.at[src_v], rows_v, sem)

    def _drain(src_v, dst_v, rows_v, sem):
        pltpu.make_async_copy(zzw_hbm.at[src_v], rows_v, sem).wait()
        pltpu.sync_copy(rows_v, acc_s.at[dst_v], add=True)

    _fetch(0, src_a, dst_a, rows_a, sem_a)

    def _chunk2(k2, carry):
        _fetch(2 * k2 + 1, src_b, dst_b, rows_b, sem_b)
        _drain(src_a, dst_a, rows_a, sem_a)
        _fetch(2 * k2 + 2, src_a, dst_a, rows_a, sem_a)
        _drain(src_b, dst_b, rows_b, sem_b)
        return carry

    lax.fori_loop(0, (_NCH - 1) // 2, _chunk2, None)
    _drain(src_a, dst_a, rows_a, sem_a)
    plsc.subcore_barrier()
    pltpu.sync_copy(acc_s.at[pl.ds(row0, _RPT)],
                    ug_hbm.at[pl.ds(c * _NP + row0, _RPT)])


def _ug_edge(zzw, zeros, src, dst):
    kern = pl.kernel(
        _ug_body,
        out_type=[jax.ShapeDtypeStruct((2 * _NP, _D), _f32)],
        mesh=_mesh,
        scratch_types=[
            pltpu.VMEM((_CH,), _i32),
            pltpu.VMEM((_CH,), _i32),
            pltpu.VMEM((_CH, _D), _f32),
            pltpu.VMEM((_CH,), _i32),
            pltpu.VMEM((_CH,), _i32),
            pltpu.VMEM((_CH, _D), _f32),
            pltpu.VMEM_SHARED((_NP, _D), _f32),
            pltpu.SemaphoreType.DMA,
            pltpu.SemaphoreType.DMA,
        ],
    )
    return kern(zzw, zeros, src, dst)


def _sd_body(ewr_hbm, zeros_hbm, src_hbm, dst_hbm, sd2_hbm, src_a, dst_a,
             wrow_a, src_b, dst_b, wrow_b, sd_v, sds_s, sem_a, sem_b):
    c = lax.axis_index("c")
    s = lax.axis_index("s")
    z16 = jnp.zeros((16,), _f32)
    iota16 = lax.iota(_i32, 16)
    o01 = jnp.where(iota16 == 1, 1.0, 0.0).astype(_f32)
    row0 = s * _RPT
    for i in range(_CH):
        for j in range(_D // 16):
            sd_v[i, pl.ds(j * 16, 16)] = z16
    pltpu.sync_copy(zeros_hbm.at[pl.ds(row0, _RPT)],
                    sds_s.at[pl.ds(row0, _RPT)])
    plsc.subcore_barrier()

    def _fetch(chunk, src_v, dst_v, wrow_v, sem):
        base = ((s * _NCH2 + chunk) * 2 + c) * _CH
        pltpu.sync_copy(src_hbm.at[pl.ds(base, _CH)], src_v)
        pltpu.sync_copy(dst_hbm.at[pl.ds(base, _CH)], dst_v)
        pltpu.async_copy(ewr_hbm.at[src_v], wrow_v, sem)

    def _drain(src_v, dst_v, wrow_v, sem):
        pltpu.make_async_copy(ewr_hbm.at[src_v], wrow_v, sem).wait()
        for r in range(_CH):
            wb = wrow_v[r, pl.ds(0, 16)]
            sd_v[r, pl.ds(0, 16)] = jnp.where(iota16 == 0, wb, o01)
        pltpu.sync_copy(sd_v, sds_s.at[dst_v], add=True)

    _fetch(0, src_a, dst_a, wrow_a, sem_a)

    def _chunk2(k2, carry):
        _fetch(2 * k2 + 1, src_b, dst_b, wrow_b, sem_b)
        _drain(src_a, dst_a, wrow_a, sem_a)
        _fetch(2 * k2 + 2, src_a, dst_a, wrow_a, sem_a)
        _drain(src_b, dst_b, wrow_b, sem_b)
        return carry

    lax.fori_loop(0, (_NCH2 - 1) // 2, _chunk2, None)
    _drain(src_a, dst_a, wrow_a, sem_a)
    plsc.subcore_barrier()
    pltpu.sync_copy(sds_s.at[pl.ds(row0, _RPT)],
                    sd2_hbm.at[pl.ds(c * _NP + row0, _RPT)])


def _sd_edge(ewr, zeros, src_p, dst_p):
    kern = pl.kernel(
        _sd_body,
        out_type=[jax.ShapeDtypeStruct((2 * _NP, _D), _f32)],
        mesh=_mesh,
        scratch_types=[
            pltpu.VMEM((_CH,), _i32),
            pltpu.VMEM((_CH,), _i32),
            pltpu.VMEM((_CH, _D), _f32),
            pltpu.VMEM((_CH,), _i32),
            pltpu.VMEM((_CH,), _i32),
            pltpu.VMEM((_CH, _D), _f32),
            pltpu.VMEM((_CH, _D), _f32),
            pltpu.VMEM_SHARED((_NP, _D), _f32),
            pltpu.SemaphoreType.DMA,
            pltpu.SemaphoreType.DMA,
        ],
    )
    return kern(ewr, zeros, src_p, dst_p)


def _comb_body(ug_ref, sd2_ref, zzw_ref, we1_ref, we2_ref, out_ref):
    sden = sd2_ref[:_N, 0:1] + sd2_ref[_NP:_NP + _N, 0:1]
    deg = sd2_ref[:_N, 1:2] + sd2_ref[_NP:_NP + _N, 1:2]
    u = ug_ref[:_N, :]
    gacc = ug_ref[_NP:_NP + _N, :]
    z = zzw_ref[_N:, :]
    pos = sden > 0.0
    h1 = jnp.where(pos, u / jnp.where(pos, sden, 1.0), 0.0)
    h2 = lax.dot_general(gacc, we1_ref[...],
                         (((1,), (1,)), ((), ())),
                         preferred_element_type=_f32)
    h2 = h2 + lax.dot_general(deg * z, we2_ref[...],
                              (((1,), (1,)), ((), ())),
                              preferred_element_type=_f32)
    out_ref[...] = h1 + h2


def _combine(ug, sd2, zzw, we1, we2):
    return pl.pallas_call(
        _comb_body,
        out_shape=jax.ShapeDtypeStruct((_N, _D), _f32),
    )(ug, sd2, zzw, we1, we2)


@jax.jit
def kernel(h, W_fc, W_attn, W_edge, edge_index):
    src = edge_index[0].astype(_i32)
    dst = edge_index[1].astype(_i32)
    src_p = jnp.concatenate([src, jnp.zeros((_EP2 - _E,), _i32)])
    dst_p = jnp.concatenate([dst, jnp.full((_EP2 - _E,), _N, _i32)])
    zeros = jnp.zeros((_NP, _D), _f32)
    zzw, ewr = _front(h, W_fc, W_attn[:, :_D])
    ug = _ug_edge(zzw, zeros, src_p, dst_p)[0]
    sd2 = _sd_edge(ewr, zeros, src_p, dst_p)[0]
    return _combine(ug, sd2, zzw, W_edge[:, :_D], W_edge[:, _D:])


# speedup vs baseline: 1.7567x; 1.1956x over previous
import jax
import jax.numpy as jnp
from jax import lax
from jax.experimental import pallas as pl
from jax.experimental.pallas import tpu as pltpu
from jax.experimental.pallas import tpu_sc as plsc

_N = 10000
_NP = 10112
_E = 320000
_EPM = 320256
_EP2 = 321024
_D = 128
_NT = 16
_CH = 48
_EPT = _EPM // _NT
_NCH = _EPT // _CH
_NCH2 = _EP2 // _CH // (2 * _NT)
_RPT = _NP // _NT

_f32 = jnp.float32
_i32 = jnp.int32

_mesh = plsc.VectorSubcoreMesh(core_axis_name="c", subcore_axis_name="s")


def _front_body(h_ref, wfc_ref, wa1_ref, zzw_ref, ewr_ref):
    z = lax.dot_general(h_ref[...], wfc_ref[...],
                        (((1,), (1,)), ((), ())),
                        preferred_element_type=_f32)
    p = lax.dot_general(z, wa1_ref[...], (((1,), (1,)), ((), ())),
                        preferred_element_type=_f32)
    ew = jnp.exp(p - jnp.max(p))
    zzw_ref[:_N, :] = z * ew
    zzw_ref[_N:, :] = z
    ewr_ref[...] = jnp.broadcast_to(ew, (_N, _D))


def _front(h, W_fc, wa1):
    return pl.pallas_call(
        _front_body,
        out_shape=[jax.ShapeDtypeStruct((2 * _N, _D), _f32),
                   jax.ShapeDtypeStruct((_N, _D), _f32)],
    )(h, W_fc, wa1)


def _ug_body(zzw_hbm, zeros_hbm, ei_hbm, ug_hbm, idx_a, src_a, dst_a,
             rows_a, idx_b, src_b, dst_b, rows_b, acc_s, sem_a, sem_b):
    c = lax.axis_index("c")
    s = lax.axis_index("s")
    row0 = s * _RPT
    pltpu.sync_copy(zeros_hbm.at[pl.ds(row0, _RPT)],
                    acc_s.at[pl.ds(row0, _RPT)])
    plsc.subcore_barrier()
    off = c * _N
    base0 = s * _NCH

    def _fetch(chunk, idx_v, src_v, dst_v, rows_v, sem):
        base = (base0 + chunk) * 2 * _CH
        pltpu.sync_copy(ei_hbm.at[pl.ds(base, 2 * _CH)], idx_v)
        for g in range(_CH // 16):
            src_v[pl.ds(g * 16, 16)] = idx_v[pl.ds(g * 16, 16)] + off
            dst_v[pl.ds(g * 16, 16)] = idx_v[pl.ds(_CH + g * 16, 16)]
        pltpu.async_copy(zzw_hbm.at[src_v], rows_v, sem)

    def _drain(src_v, dst_v, rows_v, sem):
        pltpu.make_async_copy(zzw_hbm.at[src_v], rows_v, sem).wait()
        pltpu.sync_copy(rows_v, acc_s.at[dst_v], add=True)

    _fetch(0, idx_a, src_a, dst_a, rows_a, sem_a)

    def _chunk2(k2, carry):
        _fetch(2 * k2 + 1, idx_b, src_b, dst_b, rows_b, sem_b)
        _drain(src_a, dst_a, rows_a, sem_a)
        _fetch(2 * k2 + 2, idx_a, src_a, dst_a, rows_a, sem_a)
        _drain(src_b, dst_b, rows_b, sem_b)
        return carry

    lax.fori_loop(0, (_NCH - 1) // 2, _chunk2, None)
    _drain(src_a, dst_a, rows_a, sem_a)
    plsc.subcore_barrier()
    pltpu.sync_copy(acc_s.at[pl.ds(row0, _RPT)],
                    ug_hbm.at[pl.ds(c * _NP + row0, _RPT)])


def _ug_edge(zzw, zeros, ei):
    kern = pl.kernel(
        _ug_body,
        out_type=[jax.ShapeDtypeStruct((2 * _NP, _D), _f32)],
        mesh=_mesh,
        scratch_types=[
            pltpu.VMEM((2 * _CH,), _i32),
            pltpu.VMEM((_CH,), _i32),
            pltpu.VMEM((_CH,), _i32),
            pltpu.VMEM((_CH, _D), _f32),
            pltpu.VMEM((2 * _CH,), _i32),
            pltpu.VMEM((_CH,), _i32),
            pltpu.VMEM((_CH,), _i32),
            pltpu.VMEM((_CH, _D), _f32),
            pltpu.VMEM_SHARED((_NP, _D), _f32),
            pltpu.SemaphoreType.DMA,
            pltpu.SemaphoreType.DMA,
        ],
    )
    return kern(zzw, zeros, ei)


def _sd_body(ewr_hbm, zeros_hbm, ei_hbm, sd2_hbm, idx_a, src_a, dst_a,
             wrow_a, idx_b, src_b, dst_b, wrow_b, sd_v, sds_s, sem_a, sem_b):
    c = lax.axis_index("c")
    s = lax.axis_index("s")
    z16 = jnp.zeros((16,), _f32)
    iota16 = lax.iota(_i32, 16)
    o01 = jnp.where(iota16 == 1, 1.0, 0.0).astype(_f32)
    row0 = s * _RPT
    for i in range(_CH):
        for j in range(_D // 16):
            sd_v[i, pl.ds(j * 16, 16)] = z16
    pltpu.sync_copy(zeros_hbm.at[pl.ds(row0, _RPT)],
                    sds_s.at[pl.ds(row0, _RPT)])
    plsc.subcore_barrier()

    def _fetch(chunk, idx_v, src_v, dst_v, wrow_v, sem):
        base = ((s * _NCH2 + chunk) * 2 + c) * 2 * _CH
        pltpu.sync_copy(ei_hbm.at[pl.ds(base, 2 * _CH)], idx_v)
        for g in range(_CH // 16):
            src_v[pl.ds(g * 16, 16)] = idx_v[pl.ds(g * 16, 16)]
            dst_v[pl.ds(g * 16, 16)] = idx_v[pl.ds(_CH + g * 16, 16)]
        pltpu.async_copy(ewr_hbm.at[src_v], wrow_v, sem)

    def _drain(src_v, dst_v, wrow_v, sem):
        pltpu.make_async_copy(ewr_hbm.at[src_v], wrow_v, sem).wait()
        for r in range(_CH):
            wb = wrow_v[r, pl.ds(0, 16)]
            sd_v[r, pl.ds(0, 16)] = jnp.where(iota16 == 0, wb, o01)
        pltpu.sync_copy(sd_v, sds_s.at[dst_v], add=True)

    _fetch(0, idx_a, src_a, dst_a, wrow_a, sem_a)

    def _chunk2(k2, carry):
        _fetch(2 * k2 + 1, idx_b, src_b, dst_b, wrow_b, sem_b)
        _drain(src_a, dst_a, wrow_a, sem_a)
        _fetch(2 * k2 + 2, idx_a, src_a, dst_a, wrow_a, sem_a)
        _drain(src_b, dst_b, wrow_b, sem_b)
        return carry

    lax.fori_loop(0, (_NCH2 - 1) // 2, _chunk2, None)
    _drain(src_a, dst_a, wrow_a, sem_a)
    plsc.subcore_barrier()
    pltpu.sync_copy(sds_s.at[pl.ds(row0, _RPT)],
                    sd2_hbm.at[pl.ds(c * _NP + row0, _RPT)])


def _sd_edge(ewr, zeros, ei):
    kern = pl.kernel(
        _sd_body,
        out_type=[jax.ShapeDtypeStruct((2 * _NP, _D), _f32)],
        mesh=_mesh,
        scratch_types=[
            pltpu.VMEM((2 * _CH,), _i32),
            pltpu.VMEM((_CH,), _i32),
            pltpu.VMEM((_CH,), _i32),
            pltpu.VMEM((_CH, _D), _f32),
            pltpu.VMEM((2 * _CH,), _i32),
            pltpu.VMEM((_CH,), _i32),
            pltpu.VMEM((_CH,), _i32),
            pltpu.VMEM((_CH, _D), _f32),
            pltpu.VMEM((_CH, _D), _f32),
            pltpu.VMEM_SHARED((_NP, _D), _f32),
            pltpu.SemaphoreType.DMA,
            pltpu.SemaphoreType.DMA,
        ],
    )
    return kern(ewr, zeros, ei)


def _comb_body(ug_ref, sd2_ref, zzw_ref, we1_ref, we2_ref, out_ref):
    sden = sd2_ref[:_N, 0:1] + sd2_ref[_NP:_NP + _N, 0:1]
    deg = sd2_ref[:_N, 1:2] + sd2_ref[_NP:_NP + _N, 1:2]
    u = ug_ref[:_N, :]
    gacc = ug_ref[_NP:_NP + _N, :]
    z = zzw_ref[_N:, :]
    pos = sden > 0.0
    h1 = jnp.where(pos, u / jnp.where(pos, sden, 1.0), 0.0)
    h2 = lax.dot_general(gacc, we1_ref[...],
                         (((1,), (1,)), ((), ())),
                         preferred_element_type=_f32)
    h2 = h2 + lax.dot_general(deg * z, we2_ref[...],
                              (((1,), (1,)), ((), ())),
                              preferred_element_type=_f32)
    out_ref[...] = h1 + h2


def _combine(ug, sd2, zzw, we1, we2):
    return pl.pallas_call(
        _comb_body,
        out_shape=jax.ShapeDtypeStruct((_N, _D), _f32),
    )(ug, sd2, zzw, we1, we2)


@jax.jit
def kernel(h, W_fc, W_attn, W_edge, edge_index):
    src = edge_index[0].astype(_i32)
    dst = edge_index[1].astype(_i32)
    src_p = jnp.concatenate([src, jnp.zeros((_EP2 - _E,), _i32)])
    dst_p = jnp.concatenate([dst, jnp.full((_EP2 - _E,), _N, _i32)])
    ei = jnp.stack([src_p.reshape(-1, _CH), dst_p.reshape(-1, _CH)],
                   axis=1).reshape(-1)
    zeros = jnp.zeros((_NP, _D), _f32)
    zzw, ewr = _front(h, W_fc, W_attn[:, :_D])
    ug = _ug_edge(zzw, zeros, ei)[0]
    sd2 = _sd_edge(ewr, zeros, ei)[0]
    return _combine(ug, sd2, zzw, W_edge[:, :_D], W_edge[:, _D:])


# speedup vs baseline: 2.1712x; 1.2360x over previous
import jax
import jax.numpy as jnp
from jax import lax
from jax.experimental import pallas as pl
from jax.experimental.pallas import tpu as pltpu
from jax.experimental.pallas import tpu_sc as plsc

_N = 10000
_NP = 10112
_E = 320000
_EPM = 320256
_EP2 = 321024
_D = 128
_NT = 16
_CH = 48
_EPT = _EPM // _NT
_NCH = _EPT // _CH
_NCH2 = _EP2 // _CH // (2 * _NT)
_RPT = _NP // _NT

_f32 = jnp.float32
_i32 = jnp.int32

_mesh = plsc.VectorSubcoreMesh(core_axis_name="c", subcore_axis_name="s")


def _front_body(h_ref, wfc_ref, wa1_ref, zzw_ref, ewr_ref):
    z = lax.dot_general(h_ref[...], wfc_ref[...],
                        (((1,), (1,)), ((), ())),
                        preferred_element_type=_f32)
    p = lax.dot_general(z, wa1_ref[...], (((1,), (1,)), ((), ())),
                        preferred_element_type=_f32)
    ew = jnp.exp(p - jnp.max(p))
    zzw_ref[:_N, :] = z * ew
    zzw_ref[_N:, :] = z
    ewr_ref[...] = jnp.broadcast_to(ew, (_N, _D))


def _front(h, W_fc, wa1):
    return pl.pallas_call(
        _front_body,
        out_shape=[jax.ShapeDtypeStruct((2 * _N, _D), _f32),
                   jax.ShapeDtypeStruct((_N, _D), _f32)],
    )(h, W_fc, wa1)


def _ug_body(zzw_hbm, zeros_hbm, ei_hbm, ug_hbm, idx_a, src_a, dst_a,
             rows_a, idx_b, src_b, dst_b, rows_b, acc_s, sem_a, sem_b,
             isem_a, isem_b):
    c = lax.axis_index("c")
    s = lax.axis_index("s")
    row0 = s * _RPT
    pltpu.sync_copy(zeros_hbm.at[pl.ds(row0, _RPT)],
                    acc_s.at[pl.ds(row0, _RPT)])
    plsc.subcore_barrier()
    off = c * _N
    base0 = s * _NCH

    def _fs(chunk, idx_v, isem):
        base = (base0 + chunk) * 2 * _CH
        pltpu.async_copy(ei_hbm.at[pl.ds(base, 2 * _CH)], idx_v, isem)

    def _ff(chunk, idx_v, src_v, dst_v, rows_v, sem, isem):
        base = (base0 + chunk) * 2 * _CH
        pltpu.make_async_copy(ei_hbm.at[pl.ds(base, 2 * _CH)], idx_v,
                              isem).wait()
        for g in range(_CH // 16):
            src_v[pl.ds(g * 16, 16)] = idx_v[pl.ds(g * 16, 16)] + off
            dst_v[pl.ds(g * 16, 16)] = idx_v[pl.ds(_CH + g * 16, 16)]
        pltpu.async_copy(zzw_hbm.at[src_v], rows_v, sem)

    def _drain(src_v, dst_v, rows_v, sem):
        pltpu.make_async_copy(zzw_hbm.at[src_v], rows_v, sem).wait()
        pltpu.sync_copy(rows_v, acc_s.at[dst_v], add=True)

    _fs(0, idx_a, isem_a)
    _ff(0, idx_a, src_a, dst_a, rows_a, sem_a, isem_a)
    _fs(1, idx_b, isem_b)

    def _chunk2(k2, carry):
        _ff(2 * k2 + 1, idx_b, src_b, dst_b, rows_b, sem_b, isem_b)
        _fs(2 * k2 + 2, idx_a, isem_a)
        _drain(src_a, dst_a, rows_a, sem_a)
        _ff(2 * k2 + 2, idx_a, src_a, dst_a, rows_a, sem_a, isem_a)
        _fs(2 * k2 + 3, idx_b, isem_b)
        _drain(src_b, dst_b, rows_b, sem_b)
        return carry

    lax.fori_loop(0, (_NCH - 1) // 2, _chunk2, None)
    _drain(src_a, dst_a, rows_a, sem_a)
    pltpu.make_async_copy(
        ei_hbm.at[pl.ds((base0 + _NCH) * 2 * _CH, 2 * _CH)], idx_b,
        isem_b).wait()
    plsc.subcore_barrier()
    pltpu.sync_copy(acc_s.at[pl.ds(row0, _RPT)],
                    ug_hbm.at[pl.ds(c * _NP + row0, _RPT)])


def _ug_edge(zzw, zeros, ei):
    kern = pl.kernel(
        _ug_body,
        out_type=[jax.ShapeDtypeStruct((2 * _NP, _D), _f32)],
        mesh=_mesh,
        scratch_types=[
            pltpu.VMEM((2 * _CH,), _i32),
            pltpu.VMEM((_CH,), _i32),
            pltpu.VMEM((_CH,), _i32),
            pltpu.VMEM((_CH, _D), _f32),
            pltpu.VMEM((2 * _CH,), _i32),
            pltpu.VMEM((_CH,), _i32),
            pltpu.VMEM((_CH,), _i32),
            pltpu.VMEM((_CH, _D), _f32),
            pltpu.VMEM_SHARED((_NP, _D), _f32),
            pltpu.SemaphoreType.DMA,
            pltpu.SemaphoreType.DMA,
            pltpu.SemaphoreType.DMA,
            pltpu.SemaphoreType.DMA,
        ],
    )
    return kern(zzw, zeros, ei)


def _sd_body(ewr_hbm, zeros_hbm, ei_hbm, sd2_hbm, idx_a, src_a, dst_a,
             wrow_a, idx_b, src_b, dst_b, wrow_b, sd_v, sds_s, sem_a, sem_b,
             isem_a, isem_b):
    c = lax.axis_index("c")
    s = lax.axis_index("s")
    z16 = jnp.zeros((16,), _f32)
    iota16 = lax.iota(_i32, 16)
    o01 = jnp.where(iota16 == 1, 1.0, 0.0).astype(_f32)
    row0 = s * _RPT
    for i in range(_CH):
        for j in range(_D // 16):
            sd_v[i, pl.ds(j * 16, 16)] = z16
    pltpu.sync_copy(zeros_hbm.at[pl.ds(row0, _RPT)],
                    sds_s.at[pl.ds(row0, _RPT)])
    plsc.subcore_barrier()

    def _fs(chunk, idx_v, isem):
        base = ((s * _NCH2 + chunk) * 2 + c) * 2 * _CH
        pltpu.async_copy(ei_hbm.at[pl.ds(base, 2 * _CH)], idx_v, isem)

    def _ff(chunk, idx_v, src_v, dst_v, wrow_v, sem, isem):
        base = ((s * _NCH2 + chunk) * 2 + c) * 2 * _CH
        pltpu.make_async_copy(ei_hbm.at[pl.ds(base, 2 * _CH)], idx_v,
                              isem).wait()
        for g in range(_CH // 16):
            src_v[pl.ds(g * 16, 16)] = idx_v[pl.ds(g * 16, 16)]
            dst_v[pl.ds(g * 16, 16)] = idx_v[pl.ds(_CH + g * 16, 16)]
        pltpu.async_copy(ewr_hbm.at[src_v], wrow_v, sem)

    def _drain(src_v, dst_v, wrow_v, sem):
        pltpu.make_async_copy(ewr_hbm.at[src_v], wrow_v, sem).wait()
        for r in range(_CH):
            wb = wrow_v[r, pl.ds(0, 16)]
            sd_v[r, pl.ds(0, 16)] = jnp.where(iota16 == 0, wb, o01)
        pltpu.sync_copy(sd_v, sds_s.at[dst_v], add=True)

    _fs(0, idx_a, isem_a)
    _ff(0, idx_a, src_a, dst_a, wrow_a, sem_a, isem_a)
    _fs(1, idx_b, isem_b)

    def _chunk2(k2, carry):
        _ff(2 * k2 + 1, idx_b, src_b, dst_b, wrow_b, sem_b, isem_b)
        _fs(2 * k2 + 2, idx_a, isem_a)
        _drain(src_a, dst_a, wrow_a, sem_a)
        _ff(2 * k2 + 2, idx_a, src_a, dst_a, wrow_a, sem_a, isem_a)
        _fs(2 * k2 + 3, idx_b, isem_b)
        _drain(src_b, dst_b, wrow_b, sem_b)
        return carry

    lax.fori_loop(0, (_NCH2 - 1) // 2, _chunk2, None)
    _drain(src_a, dst_a, wrow_a, sem_a)
    pltpu.make_async_copy(
        ei_hbm.at[pl.ds(((s * _NCH2 + _NCH2) * 2 + c) * 2 * _CH, 2 * _CH)],
        idx_b, isem_b).wait()
    plsc.subcore_barrier()
    pltpu.sync_copy(sds_s.at[pl.ds(row0, _RPT)],
                    sd2_hbm.at[pl.ds(c * _NP + row0, _RPT)])


def _sd_edge(ewr, zeros, ei):
    kern = pl.kernel(
        _sd_body,
        out_type=[jax.ShapeDtypeStruct((2 * _NP, _D), _f32)],
        mesh=_mesh,
        scratch_types=[
            pltpu.VMEM((2 * _CH,), _i32),
            pltpu.VMEM((_CH,), _i32),
            pltpu.VMEM((_CH,), _i32),
            pltpu.VMEM((_CH, _D), _f32),
            pltpu.VMEM((2 * _CH,), _i32),
            pltpu.VMEM((_CH,), _i32),
            pltpu.VMEM((_CH,), _i32),
            pltpu.VMEM((_CH, _D), _f32),
            pltpu.VMEM((_CH, _D), _f32),
            pltpu.VMEM_SHARED((_NP, _D), _f32),
            pltpu.SemaphoreType.DMA,
            pltpu.SemaphoreType.DMA,
            pltpu.SemaphoreType.DMA,
            pltpu.SemaphoreType.DMA,
        ],
    )
    return kern(ewr, zeros, ei)


def _comb_body(ug_ref, sd2_ref, zzw_ref, we1_ref, we2_ref, out_ref):
    sden = sd2_ref[:_N, 0:1] + sd2_ref[_NP:_NP + _N, 0:1]
    deg = sd2_ref[:_N, 1:2] + sd2_ref[_NP:_NP + _N, 1:2]
    u = ug_ref[:_N, :]
    gacc = ug_ref[_NP:_NP + _N, :]
    z = zzw_ref[_N:, :]
    pos = sden > 0.0
    h1 = jnp.where(pos, u / jnp.where(pos, sden, 1.0), 0.0)
    h2 = lax.dot_general(gacc, we1_ref[...],
                         (((1,), (1,)), ((), ())),
                         preferred_element_type=_f32)
    h2 = h2 + lax.dot_general(deg * z, we2_ref[...],
                              (((1,), (1,)), ((), ())),
                              preferred_element_type=_f32)
    out_ref[...] = h1 + h2


def _combine(ug, sd2, zzw, we1, we2):
    return pl.pallas_call(
        _comb_body,
        out_shape=jax.ShapeDtypeStruct((_N, _D), _f32),
    )(ug, sd2, zzw, we1, we2)


@jax.jit
def kernel(h, W_fc, W_attn, W_edge, edge_index):
    src = edge_index[0].astype(_i32)
    dst = edge_index[1].astype(_i32)
    src_p = jnp.concatenate([src, jnp.zeros((_EP2 - _E,), _i32)])
    dst_p = jnp.concatenate([dst, jnp.full((_EP2 - _E,), _N, _i32)])
    ei = jnp.stack([src_p.reshape(-1, _CH), dst_p.reshape(-1, _CH)],
                   axis=1).reshape(-1)
    ei = jnp.concatenate([ei, jnp.zeros((2 * _CH,), _i32)])
    zeros = jnp.zeros((_NP, _D), _f32)
    zzw, ewr = _front(h, W_fc, W_attn[:, :_D])
    ug = _ug_edge(zzw, zeros, ei)[0]
    sd2 = _sd_edge(ewr, zeros, ei)[0]
    return _combine(ug, sd2, zzw, W_edge[:, :_D], W_edge[:, _D:])
